# baseline (speedup 1.0000x reference)
import functools

import jax
import jax.numpy as jnp
from jax import lax
from jax.experimental import pallas as pl
from jax.experimental.pallas import tpu as pltpu
from jax.experimental.pallas import tpu_sc as plsc

N = 10000
E = 320000
D = 128
L = 6

NC = 2
NS = 16
F = D // NC
CHUNK = 128
NBUF = 2
NCHUNKS = 160
EPT = NCHUNKS * CHUNK
E_PAD = EPT * NS
ACC_ROWS = 10112
ROWS_PER_SUB_ZERO = ACC_ROWS // NS
ZFULL = ROWS_PER_SUB_ZERO // CHUNK
ZTAIL = ROWS_PER_SUB_ZERO % CHUNK
WB_FULL = 640
WB_TAIL = N - 15 * WB_FULL

_mesh = plsc.VectorSubcoreMesh(core_axis_name="c", subcore_axis_name="s")


@functools.partial(
    pl.kernel,
    out_type=jax.ShapeDtypeStruct((NC, N, F), jnp.float32),
    mesh=_mesh,
    scratch_types=[
        pltpu.VMEM_SHARED((N, F), jnp.float32),
        pltpu.VMEM_SHARED((ACC_ROWS, F), jnp.float32),
        pltpu.VMEM((NBUF, CHUNK), jnp.int32),
        pltpu.VMEM((NBUF, CHUNK), jnp.int32),
        pltpu.VMEM((NBUF, CHUNK, F), jnp.float32),
        pltpu.SemaphoreType.DMA((NBUF,)),
        pltpu.SemaphoreType.DMA((NBUF,)),
        pltpu.SemaphoreType.DMA,
    ],
    compiler_params=pltpu.CompilerParams(use_tc_tiling_on_sc=False),
)
def _sc_edge_agg(hw_hbm, src_hbm, dst_hbm, out_hbm, tbl, acc, sidx, didx,
                 rows, isem, gsem, tsem):
    c = lax.axis_index("c")
    s = lax.axis_index("s")

    def load_idx(j, b):
        pltpu.async_copy(src_hbm.at[s, j], sidx.at[b], isem.at[b])
        pltpu.async_copy(dst_hbm.at[s, j], didx.at[b], isem.at[b])

    def wait_idx(j, b):
        pltpu.make_async_copy(src_hbm.at[s, j], sidx.at[b],
                              isem.at[b]).wait()
        pltpu.make_async_copy(dst_hbm.at[s, j], didx.at[b],
                              isem.at[b]).wait()

    def start_gather(j, b):
        pltpu.async_copy(tbl.at[sidx.at[b]], rows.at[b], gsem.at[b])

    def wait_gather(b):
        pltpu.make_async_copy(tbl.at[sidx.at[b]], rows.at[b],
                              gsem.at[b]).wait()

    for t in range(NBUF):
        load_idx(t, t)

    @pl.when(s < NS - 1)
    def _():
        pltpu.async_copy(hw_hbm.at[c, pl.ds(s * WB_FULL, WB_FULL)],
                         tbl.at[pl.ds(s * WB_FULL, WB_FULL)], tsem)

    @pl.when(s == NS - 1)
    def _():
        pltpu.async_copy(hw_hbm.at[c, pl.ds(15 * WB_FULL, WB_TAIL)],
                         tbl.at[pl.ds(15 * WB_FULL, WB_TAIL)], tsem)

    @pl.loop(0, CHUNK)
    def _(r):
        @pl.loop(0, F, step=16)
        def _(col):
            rows[0, r, pl.ds(col, 16)] = jnp.zeros((16,), jnp.float32)

    zbase = s * ROWS_PER_SUB_ZERO
    for k in range(ZFULL):
        pltpu.sync_copy(rows.at[0], acc.at[pl.ds(zbase + k * CHUNK, CHUNK)])
    pltpu.sync_copy(rows.at[0, pl.ds(0, ZTAIL)],
                    acc.at[pl.ds(zbase + ZFULL * CHUNK, ZTAIL)])

    @pl.when(s < NS - 1)
    def _():
        pltpu.make_async_copy(hw_hbm.at[c, pl.ds(s * WB_FULL, WB_FULL)],
                              tbl.at[pl.ds(s * WB_FULL, WB_FULL)],
                              tsem).wait()

    @pl.when(s == NS - 1)
    def _():
        pltpu.make_async_copy(hw_hbm.at[c, pl.ds(15 * WB_FULL, WB_TAIL)],
                              tbl.at[pl.ds(15 * WB_FULL, WB_TAIL)],
                              tsem).wait()

    plsc.subcore_barrier()

    for t in range(NBUF - 1):
        wait_idx(t, t)
        start_gather(t, t)

    @pl.loop(0, NCHUNKS, step=NBUF)
    def _(j0):
        for b in range(NBUF):
            j = j0 + b
            gb = (b + NBUF - 1) % NBUF
            wait_gather(b)

            @pl.when(j + NBUF - 1 < NCHUNKS)
            def _():
                wait_idx(j + NBUF - 1, gb)
                start_gather(j + NBUF - 1, gb)

            pltpu.sync_copy(rows.at[b], acc.at[didx.at[b]], add=True)

            @pl.when(j + NBUF < NCHUNKS)
            def _():
                load_idx(j + NBUF, b)

    plsc.subcore_barrier()

    @pl.when(s < NS - 1)
    def _():
        pltpu.sync_copy(acc.at[pl.ds(s * WB_FULL, WB_FULL)],
                        out_hbm.at[c, pl.ds(s * WB_FULL, WB_FULL)])

    @pl.when(s == NS - 1)
    def _():
        pltpu.sync_copy(acc.at[pl.ds(15 * WB_FULL, WB_TAIL)],
                        out_hbm.at[c, pl.ds(15 * WB_FULL, WB_TAIL)])


BLK = 1000


def _tc_first_body(x_ref, w_ref, hw_ref):
    hw = lax.dot_general(
        x_ref[...], w_ref[...], (((1,), (1,)), ((), ())),
        preferred_element_type=jnp.float32)
    hw_ref[0] = hw[:, :F]
    hw_ref[1] = hw[:, F:]


def _tc_first(x, w0):
    return pl.pallas_call(
        _tc_first_body,
        grid=(N // BLK,),
        in_specs=[
            pl.BlockSpec((BLK, D), lambda i: (i, 0)),
            pl.BlockSpec((D, D), lambda i: (0, 0)),
        ],
        out_specs=pl.BlockSpec((2, BLK, F), lambda i: (0, i, 0)),
        out_shape=jax.ShapeDtypeStruct((2, N, F), jnp.float32),
    )(x, w0)


def _tc_mid_body(parts_ref, b_ref, m_ref, w_ref, hw_ref, mout_ref):
    agg = jnp.concatenate([parts_ref[0], parts_ref[1]], axis=1)
    h = jnp.maximum(agg + b_ref[...], 0.0)
    mout_ref[...] = jnp.maximum(m_ref[...], h)
    hw = lax.dot_general(
        h, w_ref[...], (((1,), (1,)), ((), ())),
        preferred_element_type=jnp.float32)
    hw_ref[0] = hw[:, :F]
    hw_ref[1] = hw[:, F:]


def _tc_mid(parts, b, m, w_next):
    return pl.pallas_call(
        _tc_mid_body,
        grid=(N // BLK,),
        in_specs=[
            pl.BlockSpec((2, BLK, F), lambda i: (0, i, 0)),
            pl.BlockSpec((1, D), lambda i: (0, 0)),
            pl.BlockSpec((BLK, D), lambda i: (i, 0)),
            pl.BlockSpec((D, D), lambda i: (0, 0)),
        ],
        out_specs=[
            pl.BlockSpec((2, BLK, F), lambda i: (0, i, 0)),
            pl.BlockSpec((BLK, D), lambda i: (i, 0)),
        ],
        out_shape=[
            jax.ShapeDtypeStruct((2, N, F), jnp.float32),
            jax.ShapeDtypeStruct((N, D), jnp.float32),
        ],
    )(parts, b.reshape(1, D), m, w_next)


def _tc_last_body(parts_ref, b_ref, m_ref, wl_ref, bl_ref, out_ref):
    agg = jnp.concatenate([parts_ref[0], parts_ref[1]], axis=1)
    h = jnp.maximum(agg + b_ref[...], 0.0)
    hmax = jnp.maximum(m_ref[...], h)
    out_ref[...] = lax.dot_general(
        hmax, wl_ref[...], (((1,), (1,)), ((), ())),
        preferred_element_type=jnp.float32) + bl_ref[...]


def _tc_last(parts, b, m, wl, bl):
    return pl.pallas_call(
        _tc_last_body,
        grid=(N // BLK,),
        in_specs=[
            pl.BlockSpec((2, BLK, F), lambda i: (0, i, 0)),
            pl.BlockSpec((1, D), lambda i: (0, 0)),
            pl.BlockSpec((BLK, D), lambda i: (i, 0)),
            pl.BlockSpec((D, D), lambda i: (0, 0)),
            pl.BlockSpec((1, D), lambda i: (0, 0)),
        ],
        out_specs=pl.BlockSpec((BLK, D), lambda i: (i, 0)),
        out_shape=jax.ShapeDtypeStruct((N, D), jnp.float32),
    )(parts, b.reshape(1, D), m, wl, bl.reshape(1, D))


def kernel(x, graph, Ws, bs, Wl, bl):
    src = graph[0]
    dst = graph[1]
    pad = E_PAD - E
    src_p = jnp.concatenate([src, jnp.zeros((pad,), jnp.int32)])
    src_p = src_p.reshape(NS, NCHUNKS, CHUNK)
    dst_p = jnp.concatenate([dst, jnp.full((pad,), N, jnp.int32)])
    dst_p = dst_p.reshape(NS, NCHUNKS, CHUNK)

    hw = _tc_first(x, Ws[0])
    m = jnp.zeros((N, D), jnp.float32)
    for i in range(L):
        parts = _sc_edge_agg(hw, src_p, dst_p)
        if i < L - 1:
            hw, m = _tc_mid(parts, bs[i], m, Ws[i + 1])
        else:
            out = _tc_last(parts, bs[i], m, Wl, bl)
    return out


# speedup vs baseline: 1.0040x; 1.0040x over previous
import functools

import jax
import jax.numpy as jnp
from jax import lax
from jax.experimental import pallas as pl
from jax.experimental.pallas import tpu as pltpu
from jax.experimental.pallas import tpu_sc as plsc

N = 10000
E = 320000
D = 128
L = 6

NC = 2
NS = 16
F = D // NC
CHUNK = 128
NBUF = 4
NCHUNKS = 160
EPT = NCHUNKS * CHUNK
E_PAD = EPT * NS
ACC_ROWS = 10112
ROWS_PER_SUB_ZERO = ACC_ROWS // NS
ZFULL = ROWS_PER_SUB_ZERO // CHUNK
ZTAIL = ROWS_PER_SUB_ZERO % CHUNK
WB_FULL = 640
WB_TAIL = N - 15 * WB_FULL

_mesh = plsc.VectorSubcoreMesh(core_axis_name="c", subcore_axis_name="s")


@functools.partial(
    pl.kernel,
    out_type=jax.ShapeDtypeStruct((NC, N, F), jnp.float32),
    mesh=_mesh,
    scratch_types=[
        pltpu.VMEM_SHARED((N, F), jnp.float32),
        pltpu.VMEM_SHARED((ACC_ROWS, F), jnp.float32),
        pltpu.VMEM((NBUF, CHUNK), jnp.int32),
        pltpu.VMEM((NBUF, CHUNK), jnp.int32),
        pltpu.VMEM((NBUF, CHUNK, F), jnp.float32),
        pltpu.SemaphoreType.DMA((NBUF,)),
        pltpu.SemaphoreType.DMA((NBUF,)),
        pltpu.SemaphoreType.DMA,
    ],
    compiler_params=pltpu.CompilerParams(use_tc_tiling_on_sc=False),
)
def _sc_edge_agg(hw_hbm, src_hbm, dst_hbm, out_hbm, tbl, acc, sidx, didx,
                 rows, isem, gsem, tsem):
    c = lax.axis_index("c")
    s = lax.axis_index("s")

    def load_idx(j, b):
        pltpu.async_copy(src_hbm.at[s, j], sidx.at[b], isem.at[b])
        pltpu.async_copy(dst_hbm.at[s, j], didx.at[b], isem.at[b])

    def wait_idx(j, b):
        pltpu.make_async_copy(src_hbm.at[s, j], sidx.at[b],
                              isem.at[b]).wait()
        pltpu.make_async_copy(dst_hbm.at[s, j], didx.at[b],
                              isem.at[b]).wait()

    def start_gather(j, b):
        pltpu.async_copy(tbl.at[sidx.at[b]], rows.at[b], gsem.at[b])

    def wait_gather(b):
        pltpu.make_async_copy(tbl.at[sidx.at[b]], rows.at[b],
                              gsem.at[b]).wait()

    for t in range(NBUF):
        load_idx(t, t)

    @pl.when(s < NS - 1)
    def _():
        pltpu.async_copy(hw_hbm.at[c, pl.ds(s * WB_FULL, WB_FULL)],
                         tbl.at[pl.ds(s * WB_FULL, WB_FULL)], tsem)

    @pl.when(s == NS - 1)
    def _():
        pltpu.async_copy(hw_hbm.at[c, pl.ds(15 * WB_FULL, WB_TAIL)],
                         tbl.at[pl.ds(15 * WB_FULL, WB_TAIL)], tsem)

    @pl.loop(0, CHUNK)
    def _(r):
        @pl.loop(0, F, step=16)
        def _(col):
            rows[0, r, pl.ds(col, 16)] = jnp.zeros((16,), jnp.float32)

    zbase = s * ROWS_PER_SUB_ZERO
    for k in range(ZFULL):
        pltpu.sync_copy(rows.at[0], acc.at[pl.ds(zbase + k * CHUNK, CHUNK)])
    pltpu.sync_copy(rows.at[0, pl.ds(0, ZTAIL)],
                    acc.at[pl.ds(zbase + ZFULL * CHUNK, ZTAIL)])

    @pl.when(s < NS - 1)
    def _():
        pltpu.make_async_copy(hw_hbm.at[c, pl.ds(s * WB_FULL, WB_FULL)],
                              tbl.at[pl.ds(s * WB_FULL, WB_FULL)],
                              tsem).wait()

    @pl.when(s == NS - 1)
    def _():
        pltpu.make_async_copy(hw_hbm.at[c, pl.ds(15 * WB_FULL, WB_TAIL)],
                              tbl.at[pl.ds(15 * WB_FULL, WB_TAIL)],
                              tsem).wait()

    plsc.subcore_barrier()

    for t in range(NBUF - 1):
        wait_idx(t, t)
        start_gather(t, t)

    @pl.loop(0, NCHUNKS, step=NBUF)
    def _(j0):
        for b in range(NBUF):
            j = j0 + b
            gb = (b + NBUF - 1) % NBUF
            wait_gather(b)

            @pl.when(j + NBUF - 1 < NCHUNKS)
            def _():
                wait_idx(j + NBUF - 1, gb)
                start_gather(j + NBUF - 1, gb)

            pltpu.sync_copy(rows.at[b], acc.at[didx.at[b]], add=True)

            @pl.when(j + NBUF < NCHUNKS)
            def _():
                load_idx(j + NBUF, b)

    plsc.subcore_barrier()

    @pl.when(s < NS - 1)
    def _():
        pltpu.sync_copy(acc.at[pl.ds(s * WB_FULL, WB_FULL)],
                        out_hbm.at[c, pl.ds(s * WB_FULL, WB_FULL)])

    @pl.when(s == NS - 1)
    def _():
        pltpu.sync_copy(acc.at[pl.ds(15 * WB_FULL, WB_TAIL)],
                        out_hbm.at[c, pl.ds(15 * WB_FULL, WB_TAIL)])


BLK = 1000


def _tc_first_body(x_ref, w_ref, hw_ref):
    hw = lax.dot_general(
        x_ref[...], w_ref[...], (((1,), (1,)), ((), ())),
        preferred_element_type=jnp.float32)
    hw_ref[0] = hw[:, :F]
    hw_ref[1] = hw[:, F:]


def _tc_first(x, w0):
    return pl.pallas_call(
        _tc_first_body,
        grid=(N // BLK,),
        in_specs=[
            pl.BlockSpec((BLK, D), lambda i: (i, 0)),
            pl.BlockSpec((D, D), lambda i: (0, 0)),
        ],
        out_specs=pl.BlockSpec((2, BLK, F), lambda i: (0, i, 0)),
        out_shape=jax.ShapeDtypeStruct((2, N, F), jnp.float32),
    )(x, w0)


def _tc_mid_body(parts_ref, b_ref, m_ref, w_ref, hw_ref, mout_ref):
    agg = jnp.concatenate([parts_ref[0], parts_ref[1]], axis=1)
    h = jnp.maximum(agg + b_ref[...], 0.0)
    mout_ref[...] = jnp.maximum(m_ref[...], h)
    hw = lax.dot_general(
        h, w_ref[...], (((1,), (1,)), ((), ())),
        preferred_element_type=jnp.float32)
    hw_ref[0] = hw[:, :F]
    hw_ref[1] = hw[:, F:]


def _tc_mid(parts, b, m, w_next):
    return pl.pallas_call(
        _tc_mid_body,
        grid=(N // BLK,),
        in_specs=[
            pl.BlockSpec((2, BLK, F), lambda i: (0, i, 0)),
            pl.BlockSpec((1, D), lambda i: (0, 0)),
            pl.BlockSpec((BLK, D), lambda i: (i, 0)),
            pl.BlockSpec((D, D), lambda i: (0, 0)),
        ],
        out_specs=[
            pl.BlockSpec((2, BLK, F), lambda i: (0, i, 0)),
            pl.BlockSpec((BLK, D), lambda i: (i, 0)),
        ],
        out_shape=[
            jax.ShapeDtypeStruct((2, N, F), jnp.float32),
            jax.ShapeDtypeStruct((N, D), jnp.float32),
        ],
    )(parts, b.reshape(1, D), m, w_next)


def _tc_last_body(parts_ref, b_ref, m_ref, wl_ref, bl_ref, out_ref):
    agg = jnp.concatenate([parts_ref[0], parts_ref[1]], axis=1)
    h = jnp.maximum(agg + b_ref[...], 0.0)
    hmax = jnp.maximum(m_ref[...], h)
    out_ref[...] = lax.dot_general(
        hmax, wl_ref[...], (((1,), (1,)), ((), ())),
        preferred_element_type=jnp.float32) + bl_ref[...]


def _tc_last(parts, b, m, wl, bl):
    return pl.pallas_call(
        _tc_last_body,
        grid=(N // BLK,),
        in_specs=[
            pl.BlockSpec((2, BLK, F), lambda i: (0, i, 0)),
            pl.BlockSpec((1, D), lambda i: (0, 0)),
            pl.BlockSpec((BLK, D), lambda i: (i, 0)),
            pl.BlockSpec((D, D), lambda i: (0, 0)),
            pl.BlockSpec((1, D), lambda i: (0, 0)),
        ],
        out_specs=pl.BlockSpec((BLK, D), lambda i: (i, 0)),
        out_shape=jax.ShapeDtypeStruct((N, D), jnp.float32),
    )(parts, b.reshape(1, D), m, wl, bl.reshape(1, D))


def kernel(x, graph, Ws, bs, Wl, bl):
    src = graph[0]
    dst = graph[1]
    pad = E_PAD - E
    src_p = jnp.concatenate([src, jnp.zeros((pad,), jnp.int32)])
    src_p = src_p.reshape(NS, NCHUNKS, CHUNK)
    dst_p = jnp.concatenate([dst, jnp.full((pad,), N, jnp.int32)])
    dst_p = dst_p.reshape(NS, NCHUNKS, CHUNK)

    hw = _tc_first(x, Ws[0])
    m = jnp.zeros((N, D), jnp.float32)
    for i in range(L):
        parts = _sc_edge_agg(hw, src_p, dst_p)
        if i < L - 1:
            hw, m = _tc_mid(parts, bs[i], m, Ws[i + 1])
        else:
            out = _tc_last(parts, bs[i], m, Wl, bl)
    return out


# speedup vs baseline: 1.0894x; 1.0850x over previous
import functools

import jax
import jax.numpy as jnp
from jax import lax
from jax.experimental import pallas as pl
from jax.experimental.pallas import tpu as pltpu
from jax.experimental.pallas import tpu_sc as plsc

N = 10000
E = 320000
D = 128
L = 6

NC = 2
NS = 16
F = D // NC
CHUNK = 256
NBUF = 2
NCHUNKS = 80
EPT = NCHUNKS * CHUNK
E_PAD = EPT * NS
ACC_ROWS = 10112
ROWS_PER_SUB_ZERO = ACC_ROWS // NS
ZFULL = ROWS_PER_SUB_ZERO // CHUNK
ZTAIL = ROWS_PER_SUB_ZERO % CHUNK
WB_FULL = 640
WB_TAIL = N - 15 * WB_FULL

_mesh = plsc.VectorSubcoreMesh(core_axis_name="c", subcore_axis_name="s")


@functools.partial(
    pl.kernel,
    out_type=jax.ShapeDtypeStruct((NC, N, F), jnp.float32),
    mesh=_mesh,
    scratch_types=[
        pltpu.VMEM_SHARED((N, F), jnp.float32),
        pltpu.VMEM_SHARED((ACC_ROWS, F), jnp.float32),
        pltpu.VMEM((NBUF, CHUNK), jnp.int32),
        pltpu.VMEM((NBUF, CHUNK), jnp.int32),
        pltpu.VMEM((NBUF, CHUNK, F), jnp.float32),
        pltpu.SemaphoreType.DMA((NBUF,)),
        pltpu.SemaphoreType.DMA((NBUF,)),
        pltpu.SemaphoreType.DMA,
    ],
    compiler_params=pltpu.CompilerParams(use_tc_tiling_on_sc=False),
)
def _sc_edge_agg(hw_hbm, src_hbm, dst_hbm, out_hbm, tbl, acc, sidx, didx,
                 rows, isem, gsem, tsem):
    c = lax.axis_index("c")
    s = lax.axis_index("s")

    def load_idx(j, b):
        pltpu.async_copy(src_hbm.at[s, j], sidx.at[b], isem.at[b])
        pltpu.async_copy(dst_hbm.at[s, j], didx.at[b], isem.at[b])

    def wait_idx(j, b):
        pltpu.make_async_copy(src_hbm.at[s, j], sidx.at[b],
                              isem.at[b]).wait()
        pltpu.make_async_copy(dst_hbm.at[s, j], didx.at[b],
                              isem.at[b]).wait()

    def start_gather(j, b):
        pltpu.async_copy(tbl.at[sidx.at[b]], rows.at[b], gsem.at[b])

    def wait_gather(b):
        pltpu.make_async_copy(tbl.at[sidx.at[b]], rows.at[b],
                              gsem.at[b]).wait()

    for t in range(NBUF):
        load_idx(t, t)

    @pl.when(s < NS - 1)
    def _():
        pltpu.async_copy(hw_hbm.at[c, pl.ds(s * WB_FULL, WB_FULL)],
                         tbl.at[pl.ds(s * WB_FULL, WB_FULL)], tsem)

    @pl.when(s == NS - 1)
    def _():
        pltpu.async_copy(hw_hbm.at[c, pl.ds(15 * WB_FULL, WB_TAIL)],
                         tbl.at[pl.ds(15 * WB_FULL, WB_TAIL)], tsem)

    @pl.loop(0, CHUNK)
    def _(r):
        @pl.loop(0, F, step=16)
        def _(col):
            rows[0, r, pl.ds(col, 16)] = jnp.zeros((16,), jnp.float32)

    zbase = s * ROWS_PER_SUB_ZERO
    for k in range(ZFULL):
        pltpu.sync_copy(rows.at[0], acc.at[pl.ds(zbase + k * CHUNK, CHUNK)])
    pltpu.sync_copy(rows.at[0, pl.ds(0, ZTAIL)],
                    acc.at[pl.ds(zbase + ZFULL * CHUNK, ZTAIL)])

    @pl.when(s < NS - 1)
    def _():
        pltpu.make_async_copy(hw_hbm.at[c, pl.ds(s * WB_FULL, WB_FULL)],
                              tbl.at[pl.ds(s * WB_FULL, WB_FULL)],
                              tsem).wait()

    @pl.when(s == NS - 1)
    def _():
        pltpu.make_async_copy(hw_hbm.at[c, pl.ds(15 * WB_FULL, WB_TAIL)],
                              tbl.at[pl.ds(15 * WB_FULL, WB_TAIL)],
                              tsem).wait()

    plsc.subcore_barrier()

    for t in range(NBUF - 1):
        wait_idx(t, t)
        start_gather(t, t)

    @pl.loop(0, NCHUNKS, step=NBUF)
    def _(j0):
        for b in range(NBUF):
            j = j0 + b
            gb = (b + NBUF - 1) % NBUF
            wait_gather(b)

            @pl.when(j + NBUF - 1 < NCHUNKS)
            def _():
                wait_idx(j + NBUF - 1, gb)
                start_gather(j + NBUF - 1, gb)

            pltpu.sync_copy(rows.at[b], acc.at[didx.at[b]], add=True)

            @pl.when(j + NBUF < NCHUNKS)
            def _():
                load_idx(j + NBUF, b)

    plsc.subcore_barrier()

    @pl.when(s < NS - 1)
    def _():
        pltpu.sync_copy(acc.at[pl.ds(s * WB_FULL, WB_FULL)],
                        out_hbm.at[c, pl.ds(s * WB_FULL, WB_FULL)])

    @pl.when(s == NS - 1)
    def _():
        pltpu.sync_copy(acc.at[pl.ds(15 * WB_FULL, WB_TAIL)],
                        out_hbm.at[c, pl.ds(15 * WB_FULL, WB_TAIL)])


BLK = 1000


def _tc_first_body(x_ref, w_ref, hw_ref):
    hw = lax.dot_general(
        x_ref[...], w_ref[...], (((1,), (1,)), ((), ())),
        preferred_element_type=jnp.float32)
    hw_ref[0] = hw[:, :F]
    hw_ref[1] = hw[:, F:]


def _tc_first(x, w0):
    return pl.pallas_call(
        _tc_first_body,
        grid=(N // BLK,),
        in_specs=[
            pl.BlockSpec((BLK, D), lambda i: (i, 0)),
            pl.BlockSpec((D, D), lambda i: (0, 0)),
        ],
        out_specs=pl.BlockSpec((2, BLK, F), lambda i: (0, i, 0)),
        out_shape=jax.ShapeDtypeStruct((2, N, F), jnp.float32),
    )(x, w0)


def _tc_mid_body(parts_ref, b_ref, m_ref, w_ref, hw_ref, mout_ref):
    agg = jnp.concatenate([parts_ref[0], parts_ref[1]], axis=1)
    h = jnp.maximum(agg + b_ref[...], 0.0)
    mout_ref[...] = jnp.maximum(m_ref[...], h)
    hw = lax.dot_general(
        h, w_ref[...], (((1,), (1,)), ((), ())),
        preferred_element_type=jnp.float32)
    hw_ref[0] = hw[:, :F]
    hw_ref[1] = hw[:, F:]


def _tc_mid(parts, b, m, w_next):
    return pl.pallas_call(
        _tc_mid_body,
        grid=(N // BLK,),
        in_specs=[
            pl.BlockSpec((2, BLK, F), lambda i: (0, i, 0)),
            pl.BlockSpec((1, D), lambda i: (0, 0)),
            pl.BlockSpec((BLK, D), lambda i: (i, 0)),
            pl.BlockSpec((D, D), lambda i: (0, 0)),
        ],
        out_specs=[
            pl.BlockSpec((2, BLK, F), lambda i: (0, i, 0)),
            pl.BlockSpec((BLK, D), lambda i: (i, 0)),
        ],
        out_shape=[
            jax.ShapeDtypeStruct((2, N, F), jnp.float32),
            jax.ShapeDtypeStruct((N, D), jnp.float32),
        ],
    )(parts, b.reshape(1, D), m, w_next)


def _tc_last_body(parts_ref, b_ref, m_ref, wl_ref, bl_ref, out_ref):
    agg = jnp.concatenate([parts_ref[0], parts_ref[1]], axis=1)
    h = jnp.maximum(agg + b_ref[...], 0.0)
    hmax = jnp.maximum(m_ref[...], h)
    out_ref[...] = lax.dot_general(
        hmax, wl_ref[...], (((1,), (1,)), ((), ())),
        preferred_element_type=jnp.float32) + bl_ref[...]


def _tc_last(parts, b, m, wl, bl):
    return pl.pallas_call(
        _tc_last_body,
        grid=(N // BLK,),
        in_specs=[
            pl.BlockSpec((2, BLK, F), lambda i: (0, i, 0)),
            pl.BlockSpec((1, D), lambda i: (0, 0)),
            pl.BlockSpec((BLK, D), lambda i: (i, 0)),
            pl.BlockSpec((D, D), lambda i: (0, 0)),
            pl.BlockSpec((1, D), lambda i: (0, 0)),
        ],
        out_specs=pl.BlockSpec((BLK, D), lambda i: (i, 0)),
        out_shape=jax.ShapeDtypeStruct((N, D), jnp.float32),
    )(parts, b.reshape(1, D), m, wl, bl.reshape(1, D))


def kernel(x, graph, Ws, bs, Wl, bl):
    src = graph[0]
    dst = graph[1]
    pad = E_PAD - E
    src_p = jnp.concatenate([src, jnp.zeros((pad,), jnp.int32)])
    src_p = src_p.reshape(NS, NCHUNKS, CHUNK)
    dst_p = jnp.concatenate([dst, jnp.full((pad,), N, jnp.int32)])
    dst_p = dst_p.reshape(NS, NCHUNKS, CHUNK)

    hw = _tc_first(x, Ws[0])
    m = jnp.zeros((N, D), jnp.float32)
    for i in range(L):
        parts = _sc_edge_agg(hw, src_p, dst_p)
        if i < L - 1:
            hw, m = _tc_mid(parts, bs[i], m, Ws[i + 1])
        else:
            out = _tc_last(parts, bs[i], m, Wl, bl)
    return out


# speedup vs baseline: 1.1150x; 1.0235x over previous
import functools

import jax
import jax.numpy as jnp
from jax import lax
from jax.experimental import pallas as pl
from jax.experimental.pallas import tpu as pltpu
from jax.experimental.pallas import tpu_sc as plsc

N = 10000
E = 320000
D = 128
L = 6

NC = 2
NS = 16
F = D // NC
CHUNK = 320
NBUF = 2
NCHUNKS = 64
EPT = NCHUNKS * CHUNK
E_PAD = EPT * NS
ACC_ROWS = 10112
ROWS_PER_SUB_ZERO = ACC_ROWS // NS
ZFULL = ROWS_PER_SUB_ZERO // CHUNK
ZTAIL = ROWS_PER_SUB_ZERO % CHUNK
WB_FULL = 640
WB_TAIL = N - 15 * WB_FULL

_mesh = plsc.VectorSubcoreMesh(core_axis_name="c", subcore_axis_name="s")


@functools.partial(
    pl.kernel,
    out_type=jax.ShapeDtypeStruct((NC, N, F), jnp.float32),
    mesh=_mesh,
    scratch_types=[
        pltpu.VMEM_SHARED((N, F), jnp.float32),
        pltpu.VMEM_SHARED((ACC_ROWS, F), jnp.float32),
        pltpu.VMEM((NBUF, CHUNK), jnp.int32),
        pltpu.VMEM((NBUF, CHUNK), jnp.int32),
        pltpu.VMEM((NBUF, CHUNK, F), jnp.float32),
        pltpu.SemaphoreType.DMA((NBUF,)),
        pltpu.SemaphoreType.DMA((NBUF,)),
        pltpu.SemaphoreType.DMA,
    ],
    compiler_params=pltpu.CompilerParams(use_tc_tiling_on_sc=False),
)
def _sc_edge_agg(hw_hbm, src_hbm, dst_hbm, out_hbm, tbl, acc, sidx, didx,
                 rows, isem, gsem, tsem):
    c = lax.axis_index("c")
    s = lax.axis_index("s")

    def load_idx(j, b):
        pltpu.async_copy(src_hbm.at[s, j], sidx.at[b], isem.at[b])
        pltpu.async_copy(dst_hbm.at[s, j], didx.at[b], isem.at[b])

    def wait_idx(j, b):
        pltpu.make_async_copy(src_hbm.at[s, j], sidx.at[b],
                              isem.at[b]).wait()
        pltpu.make_async_copy(dst_hbm.at[s, j], didx.at[b],
                              isem.at[b]).wait()

    def start_gather(j, b):
        pltpu.async_copy(tbl.at[sidx.at[b]], rows.at[b], gsem.at[b])

    def wait_gather(b):
        pltpu.make_async_copy(tbl.at[sidx.at[b]], rows.at[b],
                              gsem.at[b]).wait()

    for t in range(NBUF):
        load_idx(t, t)

    @pl.when(s < NS - 1)
    def _():
        pltpu.async_copy(hw_hbm.at[c, pl.ds(s * WB_FULL, WB_FULL)],
                         tbl.at[pl.ds(s * WB_FULL, WB_FULL)], tsem)

    @pl.when(s == NS - 1)
    def _():
        pltpu.async_copy(hw_hbm.at[c, pl.ds(15 * WB_FULL, WB_TAIL)],
                         tbl.at[pl.ds(15 * WB_FULL, WB_TAIL)], tsem)

    @pl.loop(0, CHUNK)
    def _(r):
        @pl.loop(0, F, step=16)
        def _(col):
            rows[0, r, pl.ds(col, 16)] = jnp.zeros((16,), jnp.float32)

    zbase = s * ROWS_PER_SUB_ZERO
    for k in range(ZFULL):
        pltpu.sync_copy(rows.at[0], acc.at[pl.ds(zbase + k * CHUNK, CHUNK)])
    pltpu.sync_copy(rows.at[0, pl.ds(0, ZTAIL)],
                    acc.at[pl.ds(zbase + ZFULL * CHUNK, ZTAIL)])

    @pl.when(s < NS - 1)
    def _():
        pltpu.make_async_copy(hw_hbm.at[c, pl.ds(s * WB_FULL, WB_FULL)],
                              tbl.at[pl.ds(s * WB_FULL, WB_FULL)],
                              tsem).wait()

    @pl.when(s == NS - 1)
    def _():
        pltpu.make_async_copy(hw_hbm.at[c, pl.ds(15 * WB_FULL, WB_TAIL)],
                              tbl.at[pl.ds(15 * WB_FULL, WB_TAIL)],
                              tsem).wait()

    plsc.subcore_barrier()

    for t in range(NBUF - 1):
        wait_idx(t, t)
        start_gather(t, t)

    @pl.loop(0, NCHUNKS, step=NBUF)
    def _(j0):
        for b in range(NBUF):
            j = j0 + b
            gb = (b + NBUF - 1) % NBUF
            wait_gather(b)

            @pl.when(j + NBUF - 1 < NCHUNKS)
            def _():
                wait_idx(j + NBUF - 1, gb)
                start_gather(j + NBUF - 1, gb)

            pltpu.sync_copy(rows.at[b], acc.at[didx.at[b]], add=True)

            @pl.when(j + NBUF < NCHUNKS)
            def _():
                load_idx(j + NBUF, b)

    plsc.subcore_barrier()

    @pl.when(s < NS - 1)
    def _():
        pltpu.sync_copy(acc.at[pl.ds(s * WB_FULL, WB_FULL)],
                        out_hbm.at[c, pl.ds(s * WB_FULL, WB_FULL)])

    @pl.when(s == NS - 1)
    def _():
        pltpu.sync_copy(acc.at[pl.ds(15 * WB_FULL, WB_TAIL)],
                        out_hbm.at[c, pl.ds(15 * WB_FULL, WB_TAIL)])


BLK = 1000


def _tc_first_body(x_ref, w_ref, hw_ref):
    hw = lax.dot_general(
        x_ref[...], w_ref[...], (((1,), (1,)), ((), ())),
        preferred_element_type=jnp.float32)
    hw_ref[0] = hw[:, :F]
    hw_ref[1] = hw[:, F:]


def _tc_first(x, w0):
    return pl.pallas_call(
        _tc_first_body,
        grid=(N // BLK,),
        in_specs=[
            pl.BlockSpec((BLK, D), lambda i: (i, 0)),
            pl.BlockSpec((D, D), lambda i: (0, 0)),
        ],
        out_specs=pl.BlockSpec((2, BLK, F), lambda i: (0, i, 0)),
        out_shape=jax.ShapeDtypeStruct((2, N, F), jnp.float32),
    )(x, w0)


def _tc_mid_body(parts_ref, b_ref, m_ref, w_ref, hw_ref, mout_ref):
    agg = jnp.concatenate([parts_ref[0], parts_ref[1]], axis=1)
    h = jnp.maximum(agg + b_ref[...], 0.0)
    mout_ref[...] = jnp.maximum(m_ref[...], h)
    hw = lax.dot_general(
        h, w_ref[...], (((1,), (1,)), ((), ())),
        preferred_element_type=jnp.float32)
    hw_ref[0] = hw[:, :F]
    hw_ref[1] = hw[:, F:]


def _tc_mid(parts, b, m, w_next):
    return pl.pallas_call(
        _tc_mid_body,
        grid=(N // BLK,),
        in_specs=[
            pl.BlockSpec((2, BLK, F), lambda i: (0, i, 0)),
            pl.BlockSpec((1, D), lambda i: (0, 0)),
            pl.BlockSpec((BLK, D), lambda i: (i, 0)),
            pl.BlockSpec((D, D), lambda i: (0, 0)),
        ],
        out_specs=[
            pl.BlockSpec((2, BLK, F), lambda i: (0, i, 0)),
            pl.BlockSpec((BLK, D), lambda i: (i, 0)),
        ],
        out_shape=[
            jax.ShapeDtypeStruct((2, N, F), jnp.float32),
            jax.ShapeDtypeStruct((N, D), jnp.float32),
        ],
    )(parts, b.reshape(1, D), m, w_next)


def _tc_last_body(parts_ref, b_ref, m_ref, wl_ref, bl_ref, out_ref):
    agg = jnp.concatenate([parts_ref[0], parts_ref[1]], axis=1)
    h = jnp.maximum(agg + b_ref[...], 0.0)
    hmax = jnp.maximum(m_ref[...], h)
    out_ref[...] = lax.dot_general(
        hmax, wl_ref[...], (((1,), (1,)), ((), ())),
        preferred_element_type=jnp.float32) + bl_ref[...]


def _tc_last(parts, b, m, wl, bl):
    return pl.pallas_call(
        _tc_last_body,
        grid=(N // BLK,),
        in_specs=[
            pl.BlockSpec((2, BLK, F), lambda i: (0, i, 0)),
            pl.BlockSpec((1, D), lambda i: (0, 0)),
            pl.BlockSpec((BLK, D), lambda i: (i, 0)),
            pl.BlockSpec((D, D), lambda i: (0, 0)),
            pl.BlockSpec((1, D), lambda i: (0, 0)),
        ],
        out_specs=pl.BlockSpec((BLK, D), lambda i: (i, 0)),
        out_shape=jax.ShapeDtypeStruct((N, D), jnp.float32),
    )(parts, b.reshape(1, D), m, wl, bl.reshape(1, D))


def kernel(x, graph, Ws, bs, Wl, bl):
    src = graph[0]
    dst = graph[1]
    pad = E_PAD - E
    src_p = jnp.concatenate([src, jnp.zeros((pad,), jnp.int32)])
    src_p = src_p.reshape(NS, NCHUNKS, CHUNK)
    dst_p = jnp.concatenate([dst, jnp.full((pad,), N, jnp.int32)])
    dst_p = dst_p.reshape(NS, NCHUNKS, CHUNK)

    hw = _tc_first(x, Ws[0])
    m = jnp.zeros((N, D), jnp.float32)
    for i in range(L):
        parts = _sc_edge_agg(hw, src_p, dst_p)
        if i < L - 1:
            hw, m = _tc_mid(parts, bs[i], m, Ws[i + 1])
        else:
            out = _tc_last(parts, bs[i], m, Wl, bl)
    return out
